# Initial kernel scaffold; baseline (speedup 1.0000x reference)
import functools

import jax
import jax.numpy as jnp
from jax import lax
from jax.experimental import pallas as pl
from jax.experimental.pallas import tpu as pltpu
from jax.experimental.pallas import tpu_sc as plsc

NC = 2
NS = 16
L = 16


def _proj_body(x_ref, w_ref, a_ref, feat_ref, eler_ref):
    feat = jnp.dot(x_ref[...], w_ref[...], preferred_element_type=jnp.float32)
    feat_ref[...] = feat
    eler_ref[...] = jnp.dot(feat, a_ref[...], preferred_element_type=jnp.float32)


def _combine_body(num_ref, den_ref, x_ref, b_ref, o_ref):
    n = num_ref[0] + num_ref[1]
    npts = x_ref.shape[0]
    d = den_ref[0, pl.ds(0, npts)] + den_ref[1, pl.ds(0, npts)]
    dcol = d[:, None]
    o_ref[...] = jnp.where(dcol > 0.0, n / dcol, 0.0) + x_ref[...] + b_ref[...]


def _edge_body(n_nodes, n_chunks, chunk, feat_h, eler_h, src_h, dst_h,
               num_h, den_h, src_v, dst_v, eler_v, w_v, rows_v, zden_v,
               num_s, den_s, sem):
    c = lax.axis_index("c")
    s = lax.axis_index("s")
    wid = c * NS + s
    rows_per = n_nodes // NS
    den_pad = den_s.shape[0]
    den_per = den_pad // NS

    zf = jnp.zeros((L,), jnp.float32)
    z16i = jnp.zeros((L,), jnp.int32)
    o16i = jnp.ones((L,), jnp.int32)

    @pl.loop(0, chunk)
    def _zero_rows(i):
        for f in range(128 // L):
            rows_v[i, pl.ds(L * f, L)] = zf

    @pl.loop(0, den_per // L)
    def _zero_den(i):
        zden_v[pl.ds(L * i, L)] = zf

    n_full = rows_per // chunk
    for q in range(n_full):
        pltpu.sync_copy(rows_v, num_s.at[pl.ds(s * rows_per + q * chunk, chunk)])
    rem = rows_per - n_full * chunk
    if rem:
        pltpu.sync_copy(rows_v.at[pl.ds(0, rem)],
                        num_s.at[pl.ds(s * rows_per + n_full * chunk, rem)])
    pltpu.sync_copy(zden_v, den_s.at[pl.ds(s * den_per, den_per)])
    plsc.subcore_barrier()

    pltpu.sync_copy(src_h.at[wid], src_v)
    pltpu.sync_copy(dst_h.at[wid], dst_v)
    pltpu.sync_copy(eler_h, eler_v)

    @pl.loop(0, n_chunks)
    def _wloop(i):
        for k in range(chunk // L):
            s16 = src_v[i, pl.ds(L * k, L)]
            d16 = dst_v[i, pl.ds(L * k, L)]
            elv = plsc.load_gather(eler_v, [s16, z16i])
            erv = plsc.load_gather(eler_v, [d16, o16i])
            t = elv + erv
            e = jnp.where(t > 0.0, t, 0.2 * t)
            w_v[i, pl.ds(L * k, L)] = jnp.exp(e)

    @pl.loop(0, n_chunks)
    def _mloop(i):
        pltpu.sync_copy(w_v.at[i], den_s.at[dst_v.at[i]], add=True)
        pltpu.async_copy(feat_h.at[src_v.at[i]], rows_v, sem).wait()

        @pl.loop(0, chunk)
        def _scale(j):
            wb = plsc.load_gather(
                w_v, [jnp.full((L,), i, jnp.int32), jnp.full((L,), j, jnp.int32)])
            for f in range(128 // L):
                rows_v[j, pl.ds(L * f, L)] = rows_v[j, pl.ds(L * f, L)] * wb

        pltpu.sync_copy(rows_v, num_s.at[dst_v.at[i]], add=True)

    plsc.subcore_barrier()

    pltpu.sync_copy(num_s.at[pl.ds(s * rows_per, rows_per)],
                    num_h.at[c, pl.ds(s * rows_per, rows_per), :])
    pltpu.sync_copy(den_s.at[pl.ds(s * den_per, den_per)],
                    den_h.at[c, pl.ds(s * den_per, den_per)])


def kernel(x, edge_index, W, attn_l, attn_r, bias):
    n, d_model = x.shape
    e = edge_index.shape[1]
    nw = NC * NS
    epw = e // nw
    chunk = 80
    n_chunks = epw // chunk
    assert e % nw == 0 and epw % chunk == 0 and n % NS == 0
    den_pad = ((n + L * NS - 1) // (L * NS)) * (L * NS)

    attn = jnp.stack([attn_l, attn_r], axis=1)
    src = edge_index[0].reshape(nw, n_chunks, chunk)
    dst = edge_index[1].reshape(nw, n_chunks, chunk)

    feat, eler = pl.pallas_call(
        _proj_body,
        out_shape=[
            jax.ShapeDtypeStruct((n, d_model), jnp.float32),
            jax.ShapeDtypeStruct((n, 2), jnp.float32),
        ],
    )(x, W, attn)

    mesh = plsc.VectorSubcoreMesh(
        core_axis_name="c", subcore_axis_name="s",
        num_cores=NC, num_subcores=NS)
    edge_kernel = pl.kernel(
        functools.partial(_edge_body, n, n_chunks, chunk),
        out_type=[
            jax.ShapeDtypeStruct((NC, n, d_model), jnp.float32),
            jax.ShapeDtypeStruct((NC, den_pad), jnp.float32),
        ],
        mesh=mesh,
        scratch_types=[
            pltpu.VMEM((n_chunks, chunk), jnp.int32),
            pltpu.VMEM((n_chunks, chunk), jnp.int32),
            pltpu.VMEM((n, 2), jnp.float32),
            pltpu.VMEM((n_chunks, chunk), jnp.float32),
            pltpu.VMEM((chunk, d_model), jnp.float32),
            pltpu.VMEM((den_pad // NS,), jnp.float32),
            pltpu.VMEM_SHARED((n, d_model), jnp.float32),
            pltpu.VMEM_SHARED((den_pad,), jnp.float32),
            pltpu.SemaphoreType.DMA,
        ],
    )
    num2, den2 = edge_kernel(feat, eler, src, dst)

    out = pl.pallas_call(
        _combine_body,
        out_shape=jax.ShapeDtypeStruct((n, d_model), jnp.float32),
    )(num2, den2, x, bias)
    return out.reshape(n, 1, d_model)


# speedup vs baseline: 18.2475x; 18.2475x over previous
import functools

import jax
import jax.numpy as jnp
from jax import lax
from jax.experimental import pallas as pl
from jax.experimental.pallas import tpu as pltpu
from jax.experimental.pallas import tpu_sc as plsc

NC = 2
NS = 16
L = 16


def _proj_body(x_ref, w_ref, a_ref, feat_ref, eler_ref):
    feat = jnp.dot(x_ref[...], w_ref[...], preferred_element_type=jnp.float32)
    feat_ref[...] = feat
    eler_ref[0, :] = jnp.sum(feat * a_ref[0, :], axis=1)
    eler_ref[1, :] = jnp.sum(feat * a_ref[1, :], axis=1)


def _combine_body(num_ref, den_ref, x_ref, b_ref, o_ref):
    npts = x_ref.shape[0]
    n = num_ref[0, pl.ds(0, npts)] + num_ref[1, pl.ds(0, npts)]
    d = den_ref[0, pl.ds(0, npts)] + den_ref[1, pl.ds(0, npts)]
    dcol = d[:, None]
    o_ref[...] = jnp.where(dcol > 0.0, n / dcol, 0.0) + x_ref[...] + b_ref[...]


def _edge_body(n_chunks, chunk, feat_h, eler_h, src_h, dst_h,
               num_h, den_h, src_v, dst_v, el_v, er_v, w_v, rows_v, zden_v,
               num_s, den_s, sem):
    c = lax.axis_index("c")
    s = lax.axis_index("s")
    wid = c * NS + s
    rows_per = num_s.shape[0] // NS
    den_per = den_s.shape[0] // NS

    zf = jnp.zeros((L,), jnp.float32)

    @pl.loop(0, chunk)
    def _zero_rows(i):
        for f in range(128 // L):
            rows_v[i, pl.ds(L * f, L)] = zf

    @pl.loop(0, den_per // L)
    def _zero_den(i):
        zden_v[pl.ds(L * i, L)] = zf

    n_full = rows_per // chunk
    for q in range(n_full):
        pltpu.sync_copy(rows_v, num_s.at[pl.ds(s * rows_per + q * chunk, chunk)])
    rem = rows_per - n_full * chunk
    if rem:
        pltpu.sync_copy(rows_v.at[pl.ds(0, rem)],
                        num_s.at[pl.ds(s * rows_per + n_full * chunk, rem)])
    pltpu.sync_copy(zden_v, den_s.at[pl.ds(s * den_per, den_per)])
    plsc.subcore_barrier()

    pltpu.sync_copy(eler_h.at[0], el_v)
    pltpu.sync_copy(eler_h.at[1], er_v)

    @pl.loop(0, n_chunks)
    def _mloop(i):
        pltpu.sync_copy(src_h.at[wid, i], src_v.at[0])
        pltpu.sync_copy(dst_h.at[wid, i], dst_v.at[0])
        pltpu.async_copy(feat_h.at[src_v.at[0]], rows_v, sem).wait()
        for k in range(chunk // L):
            s16 = src_v[0, pl.ds(L * k, L)]
            d16 = dst_v[0, pl.ds(L * k, L)]
            elv = plsc.load_gather(el_v, [s16])
            erv = plsc.load_gather(er_v, [d16])
            t = elv + erv
            e = jnp.where(t > 0.0, t, 0.2 * t)
            w_v[pl.ds(L * k, L)] = jnp.exp(e)
        pltpu.sync_copy(w_v, den_s.at[dst_v.at[0]], add=True)

        @pl.loop(0, chunk)
        def _scale(j):
            wb = plsc.load_gather(w_v, [jnp.full((L,), j, jnp.int32)])
            for f in range(128 // L):
                rows_v[j, pl.ds(L * f, L)] = rows_v[j, pl.ds(L * f, L)] * wb

        pltpu.sync_copy(rows_v, num_s.at[dst_v.at[0]], add=True)

    plsc.subcore_barrier()

    pltpu.sync_copy(num_s.at[pl.ds(s * rows_per, rows_per)],
                    num_h.at[c, pl.ds(s * rows_per, rows_per), :])
    pltpu.sync_copy(den_s.at[pl.ds(s * den_per, den_per)],
                    den_h.at[c, pl.ds(s * den_per, den_per)])


def kernel(x, edge_index, W, attn_l, attn_r, bias):
    n, d_model = x.shape
    e = edge_index.shape[1]
    nw = NC * NS
    epw = e // nw
    chunk = 80
    n_chunks = epw // chunk
    assert e % nw == 0 and epw % chunk == 0
    npad = ((n + L * NS - 1) // (L * NS)) * (L * NS)
    assert (npad // NS) % chunk == 0

    attn = jnp.stack([attn_l, attn_r], axis=0)
    src = edge_index[0].reshape(nw, n_chunks, chunk)
    dst = edge_index[1].reshape(nw, n_chunks, chunk)

    feat, eler = pl.pallas_call(
        _proj_body,
        out_shape=[
            jax.ShapeDtypeStruct((n, d_model), jnp.float32),
            jax.ShapeDtypeStruct((2, n), jnp.float32),
        ],
    )(x, W, attn)

    mesh = plsc.VectorSubcoreMesh(
        core_axis_name="c", subcore_axis_name="s",
        num_cores=NC, num_subcores=NS)
    edge_kernel = pl.kernel(
        functools.partial(_edge_body, n_chunks, chunk),
        out_type=[
            jax.ShapeDtypeStruct((NC, npad, d_model), jnp.float32),
            jax.ShapeDtypeStruct((NC, npad), jnp.float32),
        ],
        mesh=mesh,
        compiler_params=pltpu.CompilerParams(needs_layout_passes=False),
        scratch_types=[
            pltpu.VMEM((1, chunk), jnp.int32),
            pltpu.VMEM((1, chunk), jnp.int32),
            pltpu.VMEM((n,), jnp.float32),
            pltpu.VMEM((n,), jnp.float32),
            pltpu.VMEM((chunk,), jnp.float32),
            pltpu.VMEM((chunk, d_model), jnp.float32),
            pltpu.VMEM((npad // NS,), jnp.float32),
            pltpu.VMEM_SHARED((npad, d_model), jnp.float32),
            pltpu.VMEM_SHARED((npad,), jnp.float32),
            pltpu.SemaphoreType.DMA,
        ],
    )
    num2, den2 = edge_kernel(feat, eler, src, dst)

    out = pl.pallas_call(
        _combine_body,
        out_shape=jax.ShapeDtypeStruct((n, d_model), jnp.float32),
    )(num2, den2, x, bias)
    return out.reshape(n, 1, d_model)


# speedup vs baseline: 24.6272x; 1.3496x over previous
import functools

import jax
import jax.numpy as jnp
from jax import lax
from jax.experimental import pallas as pl
from jax.experimental.pallas import tpu as pltpu
from jax.experimental.pallas import tpu_sc as plsc

NC = 2
NS = 16
L = 16


def _proj_body(x_ref, w_ref, a_ref, feat_ref, eler_ref):
    feat = jnp.dot(x_ref[...], w_ref[...], preferred_element_type=jnp.float32)
    feat_ref[...] = feat
    eler_ref[0, :] = jnp.sum(feat * a_ref[0, :], axis=1)
    eler_ref[1, :] = jnp.sum(feat * a_ref[1, :], axis=1)


def _combine_body(num_ref, den_ref, x_ref, b_ref, o_ref):
    npts = x_ref.shape[0]
    n = num_ref[0, pl.ds(0, npts)] + num_ref[1, pl.ds(0, npts)]
    d = den_ref[0, pl.ds(0, npts)] + den_ref[1, pl.ds(0, npts)]
    dcol = d[:, None]
    o_ref[...] = jnp.where(dcol > 0.0, n / dcol, 0.0) + x_ref[...] + b_ref[...]


def _edge_body(n_chunks, chunk, feat_h, eler_h, src_h, dst_h,
               num_h, den_h, src_v, dst_v, el_v, er_v, w_v, rows_v, zden_v,
               num_s, den_s, sem0, sem1):
    c = lax.axis_index("c")
    s = lax.axis_index("s")
    wid = c * NS + s
    rows_per = num_s.shape[0] // NS
    den_per = den_s.shape[0] // NS

    zf = jnp.zeros((L,), jnp.float32)

    @pl.loop(0, chunk)
    def _zero_rows(i):
        for f in range(128 // L):
            rows_v[i, pl.ds(L * f, L)] = zf

    @pl.loop(0, den_per // L)
    def _zero_den(i):
        zden_v[pl.ds(L * i, L)] = zf

    n_full = rows_per // chunk
    for q in range(n_full):
        pltpu.sync_copy(rows_v.at[pl.ds(0, chunk)],
                        num_s.at[pl.ds(s * rows_per + q * chunk, chunk)])
    rem = rows_per - n_full * chunk
    if rem:
        pltpu.sync_copy(rows_v.at[pl.ds(0, rem)],
                        num_s.at[pl.ds(s * rows_per + n_full * chunk, rem)])
    pltpu.sync_copy(zden_v, den_s.at[pl.ds(s * den_per, den_per)])
    plsc.subcore_barrier()

    pltpu.sync_copy(eler_h.at[0], el_v)
    pltpu.sync_copy(eler_h.at[1], er_v)

    sems = (sem0, sem1)

    def fetch(i, b):
        pltpu.sync_copy(src_h.at[wid, i], src_v.at[b])
        pltpu.sync_copy(dst_h.at[wid, i], dst_v.at[b])
        pltpu.async_copy(feat_h.at[src_v.at[b]],
                         rows_v.at[pl.ds(b * chunk, chunk)], sems[b])

    def process(i, b, prefetch_i):
        pltpu.make_async_copy(feat_h.at[src_v.at[b]],
                              rows_v.at[pl.ds(b * chunk, chunk)],
                              sems[b]).wait()
        if prefetch_i is not None:
            fetch(prefetch_i, 1 - b)
        for k in range(chunk // L):
            s16 = src_v[b, pl.ds(L * k, L)]
            d16 = dst_v[b, pl.ds(L * k, L)]
            elv = plsc.load_gather(el_v, [s16])
            erv = plsc.load_gather(er_v, [d16])
            t = elv + erv
            e = jnp.where(t > 0.0, t, 0.2 * t)
            w_v[pl.ds(L * k, L)] = jnp.exp(e)
        pltpu.sync_copy(w_v, den_s.at[dst_v.at[b]], add=True)

        @pl.loop(0, chunk)
        def _scale(j):
            wb = plsc.load_gather(w_v, [jnp.full((L,), j, jnp.int32)])
            base = b * chunk + j
            for f in range(128 // L):
                rows_v[base, pl.ds(L * f, L)] = (
                    rows_v[base, pl.ds(L * f, L)] * wb)

        pltpu.sync_copy(rows_v.at[pl.ds(b * chunk, chunk)],
                        num_s.at[dst_v.at[b]], add=True)

    assert n_chunks % 2 == 1
    fetch(0, 0)

    @pl.loop(0, (n_chunks - 1) // 2)
    def _pair(p):
        process(2 * p, 0, 2 * p + 1)
        process(2 * p + 1, 1, 2 * p + 2)

    process(n_chunks - 1, 0, None)

    plsc.subcore_barrier()

    pltpu.sync_copy(num_s.at[pl.ds(s * rows_per, rows_per)],
                    num_h.at[c, pl.ds(s * rows_per, rows_per), :])
    pltpu.sync_copy(den_s.at[pl.ds(s * den_per, den_per)],
                    den_h.at[c, pl.ds(s * den_per, den_per)])


def kernel(x, edge_index, W, attn_l, attn_r, bias):
    n, d_model = x.shape
    e = edge_index.shape[1]
    nw = NC * NS
    epw = e // nw
    chunk = 80
    n_chunks = epw // chunk
    assert e % nw == 0 and epw % chunk == 0
    npad = ((n + L * NS - 1) // (L * NS)) * (L * NS)
    assert (npad // NS) % chunk == 0

    attn = jnp.stack([attn_l, attn_r], axis=0)
    src = edge_index[0].reshape(nw, n_chunks, chunk)
    dst = edge_index[1].reshape(nw, n_chunks, chunk)

    feat, eler = pl.pallas_call(
        _proj_body,
        out_shape=[
            jax.ShapeDtypeStruct((n, d_model), jnp.float32),
            jax.ShapeDtypeStruct((2, n), jnp.float32),
        ],
    )(x, W, attn)

    mesh = plsc.VectorSubcoreMesh(
        core_axis_name="c", subcore_axis_name="s",
        num_cores=NC, num_subcores=NS)
    edge_kernel = pl.kernel(
        functools.partial(_edge_body, n_chunks, chunk),
        out_type=[
            jax.ShapeDtypeStruct((NC, npad, d_model), jnp.float32),
            jax.ShapeDtypeStruct((NC, npad), jnp.float32),
        ],
        mesh=mesh,
        compiler_params=pltpu.CompilerParams(needs_layout_passes=False),
        scratch_types=[
            pltpu.VMEM((2, chunk), jnp.int32),
            pltpu.VMEM((2, chunk), jnp.int32),
            pltpu.VMEM((n,), jnp.float32),
            pltpu.VMEM((n,), jnp.float32),
            pltpu.VMEM((chunk,), jnp.float32),
            pltpu.VMEM((2 * chunk, d_model), jnp.float32),
            pltpu.VMEM((npad // NS,), jnp.float32),
            pltpu.VMEM_SHARED((npad, d_model), jnp.float32),
            pltpu.VMEM_SHARED((npad,), jnp.float32),
            pltpu.SemaphoreType.DMA,
            pltpu.SemaphoreType.DMA,
        ],
    )
    num2, den2 = edge_kernel(feat, eler, src, dst)

    out = pl.pallas_call(
        _combine_body,
        out_shape=jax.ShapeDtypeStruct((n, d_model), jnp.float32),
    )(num2, den2, x, bias)
    return out.reshape(n, 1, d_model)


# speedup vs baseline: 27.5122x; 1.1171x over previous
import functools

import jax
import jax.numpy as jnp
from jax import lax
from jax.experimental import pallas as pl
from jax.experimental.pallas import tpu as pltpu
from jax.experimental.pallas import tpu_sc as plsc

NC = 2
NS = 16
L = 16
CH = 48
DEPTH = 4


def _proj_body(x_ref, w_ref, a_ref, feat_ref, eler_ref):
    feat = jnp.dot(x_ref[...], w_ref[...], preferred_element_type=jnp.float32)
    feat_ref[...] = feat
    eler_ref[0, :] = jnp.sum(feat * a_ref[0, :], axis=1)
    eler_ref[1, :] = jnp.sum(feat * a_ref[1, :], axis=1)


def _combine_body(num_ref, den_ref, x_ref, b_ref, o_ref):
    npts = x_ref.shape[0]
    n = num_ref[0, pl.ds(0, npts)] + num_ref[1, pl.ds(0, npts)]
    d = den_ref[0, pl.ds(0, npts)] + den_ref[1, pl.ds(0, npts)]
    dcol = d[:, None]
    o_ref[...] = jnp.where(dcol > 0.0, n / dcol, 0.0) + x_ref[...] + b_ref[...]


def _edge_body(n_chunks, feat_h, eler_h, src_h, dst_h, num_h, den_h,
               sidx, didx, el_v, er_v, w_v, rows_v, zden_v,
               num_s, den_s, gsem, scsem, isem, dsem):
    c = lax.axis_index("c")
    s = lax.axis_index("s")
    wid = c * NS + s
    rows_per = num_s.shape[0] // NS
    den_per = den_s.shape[0] // NS

    zf = jnp.zeros((L,), jnp.float32)

    @pl.loop(0, CH)
    def _zero_rows(i):
        for f in range(128 // L):
            rows_v[i, pl.ds(L * f, L)] = zf

    @pl.loop(0, den_per // L)
    def _zero_den(i):
        zden_v[pl.ds(L * i, L)] = zf

    n_full = rows_per // CH
    for q in range(n_full):
        pltpu.sync_copy(rows_v.at[pl.ds(0, CH)],
                        num_s.at[pl.ds(s * rows_per + q * CH, CH)])
    rem = rows_per - n_full * CH
    if rem:
        pltpu.sync_copy(rows_v.at[pl.ds(0, rem)],
                        num_s.at[pl.ds(s * rows_per + n_full * CH, rem)])
    pltpu.sync_copy(zden_v, den_s.at[pl.ds(s * den_per, den_per)])
    plsc.subcore_barrier()

    pltpu.sync_copy(eler_h.at[0], el_v)
    pltpu.sync_copy(eler_h.at[1], er_v)

    def rows_at(r):
        return rows_v.at[pl.ds(r * CH, CH)]

    def w_at(wr):
        return w_v.at[pl.ds(wr * CH, CH)]

    def gather_desc(r):
        return pltpu.make_async_copy(feat_h.at[sidx.at[r]], rows_at(r),
                                     gsem.at[r])

    def process(i, r, first2, pf):
        r2 = (r + 2) % DEPTH
        wr = r % 2
        if not first2:
            pltpu.make_async_copy(rows_at(r2), num_s.at[didx.at[r2]],
                                  scsem.at[r2]).wait()
            pltpu.make_async_copy(w_at(wr), den_s.at[didx.at[r2]],
                                  dsem.at[wr]).wait()
        if pf:
            pltpu.async_copy(src_h.at[wid, i + 2], sidx.at[r2], isem.at[r2])
            pltpu.async_copy(dst_h.at[wid, i + 2], didx.at[r2], isem.at[r2])
        gather_desc(r).wait()
        for k in range(CH // L):
            s16 = sidx[r, pl.ds(L * k, L)]
            d16 = didx[r, pl.ds(L * k, L)]
            t = plsc.load_gather(el_v, [s16]) + plsc.load_gather(er_v, [d16])
            e = jnp.where(t > 0.0, t, 0.2 * t)
            w_v[pl.ds(wr * CH + L * k, L)] = jnp.exp(e)
        pltpu.async_copy(w_at(wr), den_s.at[didx.at[r]], dsem.at[wr],
                         add=True)

        @pl.loop(0, CH)
        def _scale(j):
            wb = plsc.load_gather(
                w_v, [jnp.full((L,), wr * CH + j, jnp.int32)])
            row = r * CH + j
            for f in range(128 // L):
                rows_v[row, pl.ds(L * f, L)] = (
                    rows_v[row, pl.ds(L * f, L)] * wb)

        if pf:
            pltpu.make_async_copy(src_h.at[wid, i + 2], sidx.at[r2],
                                  isem.at[r2]).wait()
            pltpu.make_async_copy(dst_h.at[wid, i + 2], didx.at[r2],
                                  isem.at[r2]).wait()
            pltpu.async_copy(feat_h.at[sidx.at[r2]], rows_at(r2),
                             gsem.at[r2])
        pltpu.async_copy(rows_at(r), num_s.at[didx.at[r]], scsem.at[r],
                         add=True)

    for i0 in (0, 1):
        pltpu.sync_copy(src_h.at[wid, i0], sidx.at[i0])
        pltpu.sync_copy(dst_h.at[wid, i0], didx.at[i0])
        pltpu.async_copy(feat_h.at[sidx.at[i0]], rows_at(i0), gsem.at[i0])
    for i0 in (0, 1, 2, 3):
        process(i0, i0, i0 < 2, True)

    @pl.loop(1, (n_chunks - 2) // DEPTH)
    def _quad(h):
        for o in range(DEPTH):
            process(DEPTH * h + o, o, False, True)

    process(n_chunks - 2, 0, False, False)
    process(n_chunks - 1, 1, False, False)
    for r in (0, 1):
        pltpu.make_async_copy(rows_at(r), num_s.at[didx.at[r]],
                              scsem.at[r]).wait()
        pltpu.make_async_copy(w_at(r), den_s.at[didx.at[r]],
                              dsem.at[r]).wait()
    plsc.subcore_barrier()

    pltpu.sync_copy(num_s.at[pl.ds(s * rows_per, rows_per)],
                    num_h.at[c, pl.ds(s * rows_per, rows_per), :])
    pltpu.sync_copy(den_s.at[pl.ds(s * den_per, den_per)],
                    den_h.at[c, pl.ds(s * den_per, den_per)])


def kernel(x, edge_index, W, attn_l, attn_r, bias):
    n, d_model = x.shape
    e = edge_index.shape[1]
    nw = NC * NS
    n_chunks = ((e // nw) + CH - 1) // CH
    while (n_chunks - 2) % DEPTH:
        n_chunks += 1
    epw = n_chunks * CH
    e_pad = epw * nw
    npad = ((n + L * NS - 1) // (L * NS)) * (L * NS)
    assert (n_chunks - 2) % DEPTH == 0 and npad // NS % L == 0

    attn = jnp.stack([attn_l, attn_r], axis=0)
    src = jnp.concatenate(
        [edge_index[0], jnp.zeros((e_pad - e,), jnp.int32)])
    dst = jnp.concatenate(
        [edge_index[1], jnp.full((e_pad - e,), npad - 2, jnp.int32)])
    src = src.reshape(nw, n_chunks, CH)
    dst = dst.reshape(nw, n_chunks, CH)

    feat, eler = pl.pallas_call(
        _proj_body,
        out_shape=[
            jax.ShapeDtypeStruct((n, d_model), jnp.float32),
            jax.ShapeDtypeStruct((2, n), jnp.float32),
        ],
    )(x, W, attn)
    eler_p = jnp.pad(eler, ((0, 0), (0, npad - n)))

    mesh = plsc.VectorSubcoreMesh(
        core_axis_name="c", subcore_axis_name="s",
        num_cores=NC, num_subcores=NS)
    edge_kernel = pl.kernel(
        functools.partial(_edge_body, n_chunks),
        out_type=[
            jax.ShapeDtypeStruct((NC, npad, d_model), jnp.float32),
            jax.ShapeDtypeStruct((NC, npad), jnp.float32),
        ],
        mesh=mesh,
        compiler_params=pltpu.CompilerParams(needs_layout_passes=False),
        scratch_types=[
            pltpu.VMEM((DEPTH, CH), jnp.int32),
            pltpu.VMEM((DEPTH, CH), jnp.int32),
            pltpu.VMEM((npad,), jnp.float32),
            pltpu.VMEM((npad,), jnp.float32),
            pltpu.VMEM((2 * CH,), jnp.float32),
            pltpu.VMEM((DEPTH * CH, d_model), jnp.float32),
            pltpu.VMEM((npad // NS,), jnp.float32),
            pltpu.VMEM_SHARED((npad, d_model), jnp.float32),
            pltpu.VMEM_SHARED((npad,), jnp.float32),
            pltpu.SemaphoreType.DMA((DEPTH,)),
            pltpu.SemaphoreType.DMA((DEPTH,)),
            pltpu.SemaphoreType.DMA((DEPTH,)),
            pltpu.SemaphoreType.DMA((2,)),
        ],
    )
    num2, den2 = edge_kernel(feat, eler_p, src, dst)

    out = pl.pallas_call(
        _combine_body,
        out_shape=jax.ShapeDtypeStruct((n, d_model), jnp.float32),
    )(num2, den2, x, bias)
    return out.reshape(n, 1, d_model)
